# Initial kernel scaffold; baseline (speedup 1.0000x reference)
import jax
import jax.numpy as jnp
from jax.experimental import pallas as pl


def kernel(x, edge_index, edge_attr, repeat_unit_mask, batch, W1, as1, ad1, We1, ae1, b1, g1, be1, W2, as2, ad2, We2, ae2, b2, g2, be2, W3, as3, ad3, We3, ae3, b3):
    raise NotImplementedError("write your pallas kernel here")


# speedup vs baseline: 32.6535x; 32.6535x over previous
import functools

import jax
import jax.numpy as jnp
from jax import lax
from jax.experimental import pallas as pl
from jax.experimental.pallas import tpu as pltpu
from jax.experimental.pallas import tpu_sc as plsc

N = 10000
E = 640000
HID = 128
OUT = 64
B = 64

NC = 2
NS = 16
KH = 128
K = 2 * KH
NP = 10240
RPT = NP // NS

_mesh = functools.partial(
    plsc.VectorSubcoreMesh, core_axis_name="c", subcore_axis_name="s")


def _pad_edges(n, multiple):
    return ((n + multiple - 1) // multiple) * multiple


PE_PRE = _pad_edges(E, NC * NS * K)
EN = E + N
PE = _pad_edges(EN, NC * NS * K)


def _pre_body(d1_hbm, eap_hbm, z8_hbm, out_hbm, d_v, e_v, acc):
    c = lax.axis_index("c")
    t = lax.axis_index("s")
    pltpu.sync_copy(z8_hbm.at[pl.ds(t * RPT, RPT)], acc.at[pl.ds(t * RPT, RPT)])
    plsc.subcore_barrier()
    epw = PE_PRE // (NC * NS)
    nwin = epw // K
    base_e = (c * NS + t) * epw

    def win(g, carry):
        e0 = base_e + g * K
        pltpu.sync_copy(eap_hbm.at[pl.ds(e0, K)], e_v)
        for j in range(2):
            pltpu.sync_copy(d1_hbm.at[pl.ds(e0 + j * KH, KH)], d_v.at[j])
            pltpu.sync_copy(e_v.at[pl.ds(j * KH, KH)],
                            acc.at[d_v.at[j]], add=True)
        return carry

    lax.fori_loop(0, nwin, win, 0)
    plsc.subcore_barrier()
    pltpu.sync_copy(acc.at[pl.ds(t * RPT, RPT)],
                    out_hbm.at[c, pl.ds(t * RPT, RPT)])


def _pre_pass(dst, ea):
    pad = PE_PRE - E
    d_pad = jnp.concatenate([dst, (jnp.arange(pad, dtype=jnp.int32) % N)])
    eap = jnp.concatenate(
        [ea, jnp.ones((E, 1), jnp.float32), jnp.zeros((E, 1), jnp.float32)],
        axis=1)
    eap = jnp.concatenate([eap, jnp.zeros((pad, 8), jnp.float32)], axis=0)
    z8 = jnp.zeros((NP, 8), jnp.float32)

    fn = pl.kernel(
        _pre_body,
        out_type=jax.ShapeDtypeStruct((NC, NP, 8), jnp.float32),
        mesh=_mesh(),
        scratch_types=[
            pltpu.VMEM((2, KH), jnp.int32),
            pltpu.VMEM((K, 8), jnp.float32),
            pltpu.VMEM_SHARED((NP, 8), jnp.float32),
        ],
    )
    pre = fn(d_pad, eap, z8)
    s = pre[0, :N] + pre[1, :N]
    deg = s[:, 6]
    loop6 = s[:, :6] / jnp.maximum(deg, 1.0)[:, None]
    return loop6


def _make_edge_kernel(heads, feat, width):
    ch = feat // heads
    epw = PE // (NC * NS)
    nwin = epw // KH

    def body(s1_hbm, d1_hbm, q16_hbm, xp_hbm, adw_hbm, m16_hbm,
             zw_hbm, out_hbm,
             s_f, d_f, g_v, ad_v, q_v, m_v, acc, sem):
        c = lax.axis_index("c")
        t = lax.axis_index("s")
        pltpu.sync_copy(m16_hbm, m_v)
        pltpu.sync_copy(zw_hbm.at[pl.ds(t * RPT, RPT)],
                        acc.at[pl.ds(t * RPT, RPT)])
        plsc.subcore_barrier()
        mvec = m_v[...]
        base_e = (c * NS + t) * epw
        lanes = lax.iota(jnp.int32, 16)
        lane_ok = lanes < heads

        def win(g, carry):
            e0 = base_e + g * KH
            pltpu.sync_copy(s1_hbm.at[pl.ds(e0, KH)], s_f)
            pltpu.sync_copy(d1_hbm.at[pl.ds(e0, KH)], d_f)
            pltpu.sync_copy(q16_hbm.at[pl.ds(e0, KH)], q_v)
            pltpu.async_copy(xp_hbm.at[s_f], g_v, sem).wait()
            pltpu.async_copy(adw_hbm.at[d_f], ad_v, sem).wait()

            def ebody(e, cy):
                a1 = g_v[e, pl.ds(feat, 16)]
                al = a1 + ad_v[e] + q_v[e]
                al = jnp.where(al > 0, al, al * jnp.float32(0.2))
                pe = jnp.exp(al - mvec)
                ok = jnp.logical_and(lane_ok, e0 + e < EN)
                pe = jnp.where(ok, pe, jnp.float32(0.0))
                g_v[e, pl.ds(feat, 16)] = pe
                for h in range(heads):
                    pb = jnp.broadcast_to(pe[h], (16,))
                    for v2 in range(ch // 16):
                        col = h * ch + v2 * 16
                        g_v[e, pl.ds(col, 16)] = g_v[e, pl.ds(col, 16)] * pb
                return cy

            lax.fori_loop(0, KH, ebody, 0)
            pltpu.sync_copy(g_v, acc.at[d_f], add=True)
            return carry

        lax.fori_loop(0, nwin, win, 0)
        plsc.subcore_barrier()
        pltpu.sync_copy(acc.at[pl.ds(t * RPT, RPT)],
                        out_hbm.at[c, pl.ds(t * RPT, RPT)])

    return pl.kernel(
        body,
        out_type=jax.ShapeDtypeStruct((NC, NP, width), jnp.float32),
        mesh=_mesh(),
        compiler_params=pltpu.CompilerParams(
            needs_layout_passes=False, use_tc_tiling_on_sc=False),
        scratch_types=[
            pltpu.VMEM((KH,), jnp.int32),
            pltpu.VMEM((KH,), jnp.int32),
            pltpu.VMEM((KH, width), jnp.float32),
            pltpu.VMEM((KH, 16), jnp.float32),
            pltpu.VMEM((KH, 16), jnp.float32),
            pltpu.VMEM((16,), jnp.float32),
            pltpu.VMEM_SHARED((NP, width), jnp.float32),
            pltpu.SemaphoreType.DMA,
        ],
    )


_edge_kernel_12 = _make_edge_kernel(4, HID, 144)
_edge_kernel_3 = _make_edge_kernel(1, OUT, 80)


def _gat_layer(h_in, s1, d1, q_all, W, asrc, adst, bias, heads):
    feat = W.shape[1]
    width = 144 if heads == 4 else 80
    ch = feat // heads
    xp = h_in @ W
    xr = xp.reshape(N, heads, ch)
    asv = (xr * asrc.reshape(1, heads, ch)).sum(-1)
    adv = (xr * adst.reshape(1, heads, ch)).sum(-1)
    m = asv.max(axis=0) + adv.max(axis=0) + q_all.max(axis=0)
    m16 = jnp.zeros((16,), jnp.float32).at[:heads].set(m)

    xp_pad = jnp.zeros((NP, width), jnp.float32)
    xp_pad = xp_pad.at[:N, :feat].set(xp)
    xp_pad = xp_pad.at[:N, feat:feat + heads].set(asv)
    adw = jnp.zeros((NP, 16), jnp.float32).at[:N, :heads].set(adv)
    q16 = jnp.zeros((PE, 16), jnp.float32).at[:EN, :heads].set(q_all)
    zw = jnp.zeros((NP, width), jnp.float32)

    fn = _edge_kernel_12 if heads == 4 else _edge_kernel_3
    o = fn(s1, d1, q16, xp_pad, adw, m16, zw)
    acc = (o[0, :N] + o[1, :N])
    den = acc[:, feat:feat + heads].reshape(N, heads, 1) + 1e-16
    return (acc[:, :feat].reshape(N, heads, ch) / den).reshape(N, feat) + bias


def _bn(x, g, b):
    return x / jnp.sqrt(1.0 + 1e-5) * g + b


def _fold_ae(We, ae, heads, ch):
    return (We.reshape(6, heads, ch) * ae.reshape(1, heads, ch)).sum(-1)


def kernel(x, edge_index, edge_attr, repeat_unit_mask, batch,
           W1, as1, ad1, We1, ae1, b1, g1, be1,
           W2, as2, ad2, We2, ae2, b2, g2, be2,
           W3, as3, ad3, We3, ae3, b3):
    src, dst = edge_index[0], edge_index[1]
    loop6 = _pre_pass(dst, edge_attr)

    sl = jnp.arange(N, dtype=jnp.int32)
    pad = PE - EN
    s1 = jnp.concatenate([src, sl, jnp.zeros((pad,), jnp.int32)])
    d1 = jnp.concatenate([dst, sl, (jnp.arange(pad, dtype=jnp.int32) % N)])

    eaf = jnp.concatenate([edge_attr, loop6], axis=0)
    q1 = eaf @ _fold_ae(We1, ae1, 4, HID // 4)
    q2 = eaf @ _fold_ae(We2, ae2, 4, HID // 4)
    q3 = eaf @ _fold_ae(We3, ae3, 1, OUT)

    h = _gat_layer(x, s1, d1, q1, W1, as1, ad1, b1, 4)
    h = jax.nn.elu(_bn(h, g1, be1))
    h = _gat_layer(h, s1, d1, q2, W2, as2, ad2, b2, 4)
    h = jax.nn.elu(_bn(h, g2, be2))
    h = _gat_layer(h, s1, d1, q3, W3, as3, ad3, b3, 1)
    h = jax.nn.elu(h)

    seg = jnp.where(repeat_unit_mask == 1, batch, B)
    out = jax.ops.segment_max(h, seg, num_segments=B)
    out = jnp.where(jnp.isneginf(out), 0.0, out)
    return out
